# Initial kernel scaffold; baseline (speedup 1.0000x reference)
import jax
import jax.numpy as jnp
from jax.experimental import pallas as pl


def kernel(batch, w1, bias1, w2, bias2, w3, bias3, w4, bias4, w5, bias5, bn2_gamma, bn2_beta, bn2_mean, bn2_var, bn3_gamma, bn3_beta, bn3_mean, bn3_var, bn4_gamma, bn4_beta, bn4_mean, bn4_var, bn5_gamma, bn5_beta, bn5_mean, bn5_var, l1_w, l1_b, l2_w, l2_b):
    raise NotImplementedError("write your pallas kernel here")


# speedup vs baseline: 4867.8064x; 4867.8064x over previous
import numpy as np
import jax
import jax.numpy as jnp
from jax import lax
from jax.experimental import pallas as pl
from jax.experimental.pallas import tpu as pltpu

_N = 10
_STEP = 1.0 / (_N - 1)
_B, _E, _NC, _H, _W = 64, 2, 8, 256, 256
_V = 255
_LANES = 128
_ROWS = _H * _W // _LANES
_IPB = 4
_RB = 128


def _interp_matrix(n, step):
    mat = 4.0 * np.eye(n - 2)
    np.fill_diagonal(mat[1:, :-1], 1.0)
    np.fill_diagonal(mat[:-1, 1:], 1.0)
    A = 6.0 * np.linalg.inv(mat) / step ** 2
    z = np.zeros(n - 2)
    A = np.vstack([z, A, z])
    Bm = np.zeros([n - 2, n])
    np.fill_diagonal(Bm, 1.0)
    np.fill_diagonal(Bm[:, 1:], -2.0)
    np.fill_diagonal(Bm[:, 2:], 1.0)
    return (A @ Bm).astype(np.float32)


_MAT_T = _interp_matrix(_N, _STEP).T.copy()


def _conv(x, w, b):
    y = lax.conv_general_dilated(x, w, (2, 2), 'VALID',
                                 dimension_numbers=('NCHW', 'OIHW', 'NCHW'))
    return y + b[None, :, None, None]


def _convl(x, w, b):
    y = lax.conv_general_dilated(x, jnp.transpose(w, (2, 3, 1, 0)), (2, 2),
                                 'VALID',
                                 dimension_numbers=('NHWC', 'HWIO', 'NHWC'))
    return y + b[None, None, None, :]


def _bnl(x, g, beta, m, v):
    s = (g * lax.rsqrt(v + 1e-5))
    return (x - m[None, None, None, :]) * s[None, None, None, :] + beta[None, None, None, :]


def _bn(x, g, beta, m, v):
    s = (g * lax.rsqrt(v + 1e-5))
    return (x - m[None, :, None, None]) * s[None, :, None, None] + beta[None, :, None, None]


def _head_kernel(y_ref, w1_ref, b1_ref, w2_ref, b2_ref, matt_ref, oh_ref,
                 vf_ref, tab_ref, spl_ref):
    y = y_ref[...]
    feat = jnp.sum(y, axis=1) * jnp.float32(1.0 / 49.0)
    hh = lax.dot_general(feat, w1_ref[...], (((1,), (1,)), ((), ())),
                         preferred_element_type=jnp.float32) + b1_ref[...]
    hh = jnp.maximum(hh, 0.0)
    ys = lax.dot_general(hh, w2_ref[...], (((1,), (1,)), ((), ())),
                         preferred_element_type=jnp.float32) + b2_ref[...]
    matt = matt_ref[...]
    oh = oh_ref[...]
    vf = vf_ref[...]
    h = jnp.float32(_STEP)
    tab_ref[...] = jnp.zeros_like(tab_ref)
    for e in range(_E):
        for c in range(3):
            j = e * 3 + c
            ysj = ys[:, _N * j:_N * (j + 1)]
            iota = lax.broadcasted_iota(jnp.int32, (_B, _N), 1).astype(jnp.float32)
            ytj = ysj + iota * h
            M = jnp.dot(ytj, matt, preferred_element_type=jnp.float32)
            aj = (M[:, 1:] - M[:, :-1]) * jnp.float32(1.0 / (6.0 * _STEP))
            bj = M[:, :-1] * jnp.float32(0.5)
            cj = (ytj[:, 1:] - ytj[:, :-1]) * jnp.float32(1.0 / _STEP) \
                - (M[:, 1:] + 2.0 * M[:, :-1]) * jnp.float32(_STEP / 6.0)
            dj = ytj[:, :-1]
            tab_ref[:, j, 0:9] = aj
            tab_ref[:, j, 32:41] = bj
            tab_ref[:, j, 64:73] = cj
            tab_ref[:, j, 96:105] = dj
            ga = jnp.dot(aj, oh, preferred_element_type=jnp.float32)
            gb = jnp.dot(bj, oh, preferred_element_type=jnp.float32)
            gc = jnp.dot(cj, oh, preferred_element_type=jnp.float32)
            gd = jnp.dot(dj, oh, preferred_element_type=jnp.float32)
            spl_ref[e, :, c, :] = ((ga * vf + gb) * vf + gc) * vf + gd


def _apply_kernel(x_ref, tab_ref, o_ref):
    iota_s = lax.broadcasted_iota(jnp.int32, (8, _LANES), 0)
    for m in range(_IPB):
        for c in range(3):
            tabs = []
            for e in range(_E):
                row = tab_ref[m, e * 3 + c:e * 3 + c + 1, :]
                rowb = jnp.broadcast_to(row, (8, _LANES))
                ent = []
                for off in (0, 32, 64, 96):
                    src = jnp.take_along_axis(rowb, iota_s + off, axis=1)
                    fix = jnp.take_along_axis(
                        rowb, jnp.full((8, _LANES), off + 8, jnp.int32), axis=1)
                    ent.append((pltpu.repeat(src, _W // _LANES, axis=1),
                                pltpu.repeat(fix, _W // _LANES, axis=1)))
                tabs.append(ent)
            for rs in range(0, _H, _RB):
                x = x_ref[m, c, rs:rs + _RB, :]
                t = jnp.clip(jnp.floor(x * jnp.float32(_N - 1)), 0.0,
                             jnp.float32(_N - 2))
                xi = t.astype(jnp.int32)
                xi7 = jnp.minimum(xi, 7)
                m8 = xi == 8
                xf = x - t * jnp.float32(_STEP)
                for e in range(_E):
                    ga, gb, gc, gd = [
                        jnp.where(m8, pltpu.repeat(fix, _RB // 8, axis=0),
                                  jnp.take_along_axis(src, xi7, axis=0))
                        for (src, fix) in tabs[e]]
                    o_ref[e, m, c, rs:rs + _RB, :] = \
                        ((ga * xf + gb) * xf + gc) * xf + gd


def kernel(batch, w1, bias1, w2, bias2, w3, bias3, w4, bias4, w5, bias5,
           bn2_gamma, bn2_beta, bn2_mean, bn2_var,
           bn3_gamma, bn3_beta, bn3_mean, bn3_var,
           bn4_gamma, bn4_beta, bn4_mean, bn4_var,
           bn5_gamma, bn5_beta, bn5_mean, bn5_var,
           l1_w, l1_b, l2_w, l2_b):
    y = jax.nn.relu(_conv(batch, w1, bias1))
    y = jnp.transpose(y, (0, 2, 3, 1))
    y = _bnl(jax.nn.relu(_convl(y, w2, bias2)), bn2_gamma, bn2_beta, bn2_mean, bn2_var)
    y = _bnl(jax.nn.relu(_convl(y, w3, bias3)), bn3_gamma, bn3_beta, bn3_mean, bn3_var)
    y = _bnl(jax.nn.relu(_convl(y, w4, bias4)), bn4_gamma, bn4_beta, bn4_mean, bn4_var)
    y = _bnl(jax.nn.relu(_convl(y, w5, bias5)), bn5_gamma, bn5_beta, bn5_mean, bn5_var)
    y3 = y.reshape(_B, 49, 16 * _NC)

    vals = jnp.arange(0.0, 1.0, 1.0 / _V, dtype=jnp.float32)
    vi = jnp.clip(jnp.floor(vals / _STEP), 0, _N - 2).astype(jnp.int32)
    vf = (vals - vi.astype(jnp.float32) * _STEP)[None, :]
    oh = (vi[None, :] == jnp.arange(_N - 1, dtype=jnp.int32)[:, None]
          ).astype(jnp.float32)

    tab, splines = pl.pallas_call(
        _head_kernel,
        out_shape=(
            jax.ShapeDtypeStruct((_B, 8, _LANES), jnp.float32),
            jax.ShapeDtypeStruct((_E, _B, 3, _V), jnp.float32),
        ),
        name="spline_head",
    )(y3, l1_w, l1_b[None, :], l2_w, l2_b[None, :],
      jnp.asarray(_MAT_T), oh, vf)

    out = pl.pallas_call(
        _apply_kernel,
        grid=(_B // _IPB,),
        in_specs=[
            pl.BlockSpec((_IPB, 3, _H, _W), lambda i: (i, 0, 0, 0)),
            pl.BlockSpec((_IPB, 8, _LANES), lambda i: (i, 0, 0)),
        ],
        out_specs=pl.BlockSpec((_E, _IPB, 3, _H, _W),
                               lambda i: (0, i, 0, 0, 0)),
        out_shape=jax.ShapeDtypeStruct((_E, _B, 3, _H, _W), jnp.float32),
        compiler_params=pltpu.CompilerParams(
            dimension_semantics=("parallel",),
            vmem_limit_bytes=56 * 1024 * 1024,
        ),
        name="spline_apply",
    )(batch, tab)
    return out, splines
